# baseline (speedup 1.0000x reference)
import math

import jax
import jax.numpy as jnp
import numpy as np
from jax.experimental import pallas as pl

LMAX = 2
CHANNELS_OUT = 8
OUTPUT_SIZE = 8
NUM_BASIS = 8
MAX_RADIUS = 5.0
AVG_NEIGH = 16.0


def _fact(n):
    return math.factorial(n)


def _cg(j1, m1, j2, m2, j3, m3):
    if m1 + m2 != m3:
        return 0.0
    if j3 < abs(j1 - j2) or j3 > j1 + j2:
        return 0.0
    pre = math.sqrt((2 * j3 + 1) * _fact(j1 + j2 - j3) * _fact(j1 - j2 + j3) * _fact(-j1 + j2 + j3) / _fact(j1 + j2 + j3 + 1))
    pre *= math.sqrt(_fact(j3 + m3) * _fact(j3 - m3) * _fact(j1 - m1) * _fact(j1 + m1) * _fact(j2 - m2) * _fact(j2 + m2))
    kmin = max(0, j2 - j3 - m1, j1 - j3 + m2)
    kmax = min(j1 + j2 - j3, j1 - m1, j2 + m2)
    s = 0.0
    for k in range(kmin, kmax + 1):
        s += (-1.0) ** k / (_fact(k) * _fact(j1 + j2 - j3 - k) * _fact(j1 - m1 - k) * _fact(j2 + m2 - k) * _fact(j3 - j2 + m1 + k) * _fact(j3 - j1 - m2 + k))
    return pre * s


def _real_basis(l):
    U = np.zeros((2 * l + 1, 2 * l + 1), dtype=np.complex128)
    for m in range(-l, l + 1):
        if m < 0:
            U[m + l, m + l] = 1j / math.sqrt(2)
            U[m + l, -m + l] = -1j * (-1.0) ** abs(m) / math.sqrt(2)
        elif m == 0:
            U[l, l] = 1.0
        else:
            U[m + l, m + l] = (-1.0) ** m / math.sqrt(2)
            U[m + l, -m + l] = 1.0 / math.sqrt(2)
    return U


def _w3j_real(l1, l2, l3):
    C = np.zeros((2 * l1 + 1, 2 * l2 + 1, 2 * l3 + 1))
    for m1 in range(-l1, l1 + 1):
        for m2 in range(-l2, l2 + 1):
            for m3 in range(-l3, l3 + 1):
                C[m1 + l1, m2 + l2, m3 + l3] = _cg(l1, m1, l2, m2, l3, m3)
    U1, U2, U3 = _real_basis(l1), _real_basis(l2), _real_basis(l3)
    W = np.einsum('ai,bj,ck,ijk->abc', U1, U2, U3.conj(), C.astype(np.complex128))
    Wr, Wi = W.real, W.imag
    Wp = Wr if np.linalg.norm(Wr) >= np.linalg.norm(Wi) else Wi
    nrm = np.linalg.norm(Wp)
    if nrm > 0:
        Wp = Wp / nrm
    return Wp.astype(np.float32)


_PATHS = [(l1, l2, l3) for l1 in range(LMAX + 1) for l2 in range(LMAX + 1) for l3 in range(abs(l1 - l2), min(l1 + l2, LMAX) + 1)]
_NUM_PATHS = len(_PATHS)
_W3J = [jnp.asarray(_w3j_real(*p)) for p in _PATHS]


def _harm(n):
    x, y, z = n[:, 0], n[:, 1], n[:, 2]
    Y = [jnp.ones((n.shape[0], 1), dtype=n.dtype)]
    Y.append(math.sqrt(3.0) * jnp.stack([x, y, z], axis=-1))
    Y.append(jnp.stack([
        math.sqrt(15.0) * x * y,
        math.sqrt(15.0) * y * z,
        math.sqrt(5.0) / 2.0 * (3.0 * z * z - 1.0),
        math.sqrt(15.0) * x * z,
        math.sqrt(15.0) / 2.0 * (x * x - y * y)], axis=-1))
    return Y


def _soft_one_hot(x, start, end, number):
    values = jnp.linspace(start, end, number + 2)
    step = values[1] - values[0]
    values = values[1:-1]
    diff = (x[:, None] - values[None, :]) / step
    return jnp.exp(-diff ** 2) / 1.12


def _scale_kernel(x_ref, o_ref):
    o_ref[...] = x_ref[...] * (1.0 / AVG_NEIGH)


def kernel(pos, A, batch, edge_src, edge_dst, edge_shifts, cell, emb_table, w1, b1, w2, b2, fc_w1, fc_b1, fc_w2, fc_b2, fc_w3, fc_b3, tp_weights):
    edge_vec = pos[edge_dst] - pos[edge_src]
    edge_length = jnp.sqrt(jnp.sum(edge_vec ** 2, axis=1))
    n = edge_vec / jnp.maximum(edge_length, 1e-08)[:, None]
    Ai = jax.nn.silu(emb_table[A] @ w1 + b1) @ w2 + b2
    Y = _harm(n)
    Ai_e = Ai[edge_src]
    f_in = [Ai_e[:, :, None] * Y[l][:, None, :] for l in range(LMAX + 1)]
    emb = _soft_one_hot(edge_length, 0.0, MAX_RADIUS, NUM_BASIS) * math.sqrt(NUM_BASIS)
    h = jax.nn.silu(emb @ fc_w1 + fc_b1)
    h = jax.nn.silu(h @ fc_w2 + fc_b2)
    gates = h @ fc_w3 + fc_b3
    n_edges = edge_src.shape[0]
    out_blocks = [jnp.zeros((n_edges, CHANNELS_OUT, 2 * l + 1), dtype=pos.dtype) for l in range(LMAX + 1)]
    for p, (l1, l2, l3) in enumerate(_PATHS):
        t = jnp.einsum('eui,ej,ijk->euk', f_in[l1], Y[l2], _W3J[p])
        t = jnp.einsum('euk,uv->evk', t, tp_weights[p]) / math.sqrt(OUTPUT_SIZE)
        out_blocks[l3] = out_blocks[l3] + t * gates[:, p][:, None, None]
    parts = [out_blocks[l].reshape(n_edges, CHANNELS_OUT * (2 * l + 1)) for l in range(LMAX + 1)]
    edge_features = jnp.concatenate(parts, axis=-1)
    num_nodes = pos.shape[0]
    out = jnp.zeros((num_nodes, edge_features.shape[1]), dtype=edge_features.dtype).at[edge_dst].add(edge_features)
    return pl.pallas_call(
        _scale_kernel,
        out_shape=jax.ShapeDtypeStruct(out.shape, out.dtype),
    )(out)


# speedup vs baseline: 20.8786x; 20.8786x over previous
import functools
import math

import jax
import jax.numpy as jnp
import numpy as np
from jax import lax
from jax.experimental import pallas as pl
from jax.experimental.pallas import tpu as pltpu
from jax.experimental.pallas import tpu_sc as plsc

LMAX = 2
NUM_BASIS = 8
MAX_RADIUS = 5.0
AVG_NEIGH = 16.0
N_NODES = 10000
N_EDGES = 640000
N_PAD = 10240
E_PAD = 643072
EB = 4096
N_W = 32
CH = 128
CHUNKS = E_PAD // (N_W * CH)
ROWS_W = N_PAD // 16


def _fact(n):
    return math.factorial(n)


def _cg(j1, m1, j2, m2, j3, m3):
    if m1 + m2 != m3:
        return 0.0
    if j3 < abs(j1 - j2) or j3 > j1 + j2:
        return 0.0
    pre = math.sqrt((2 * j3 + 1) * _fact(j1 + j2 - j3) * _fact(j1 - j2 + j3) * _fact(-j1 + j2 + j3) / _fact(j1 + j2 + j3 + 1))
    pre *= math.sqrt(_fact(j3 + m3) * _fact(j3 - m3) * _fact(j1 - m1) * _fact(j1 + m1) * _fact(j2 - m2) * _fact(j2 + m2))
    kmin = max(0, j2 - j3 - m1, j1 - j3 + m2)
    kmax = min(j1 + j2 - j3, j1 - m1, j2 + m2)
    s = 0.0
    for k in range(kmin, kmax + 1):
        s += (-1.0) ** k / (_fact(k) * _fact(j1 + j2 - j3 - k) * _fact(j1 - m1 - k) * _fact(j2 + m2 - k) * _fact(j3 - j2 + m1 + k) * _fact(j3 - j1 - m2 + k))
    return pre * s


def _real_basis(l):
    U = np.zeros((2 * l + 1, 2 * l + 1), dtype=np.complex128)
    for m in range(-l, l + 1):
        if m < 0:
            U[m + l, m + l] = 1j / math.sqrt(2)
            U[m + l, -m + l] = -1j * (-1.0) ** abs(m) / math.sqrt(2)
        elif m == 0:
            U[l, l] = 1.0
        else:
            U[m + l, m + l] = (-1.0) ** m / math.sqrt(2)
            U[m + l, -m + l] = 1.0 / math.sqrt(2)
    return U


def _w3j_real(l1, l2, l3):
    C = np.zeros((2 * l1 + 1, 2 * l2 + 1, 2 * l3 + 1))
    for m1 in range(-l1, l1 + 1):
        for m2 in range(-l2, l2 + 1):
            for m3 in range(-l3, l3 + 1):
                C[m1 + l1, m2 + l2, m3 + l3] = _cg(l1, m1, l2, m2, l3, m3)
    U1, U2, U3 = _real_basis(l1), _real_basis(l2), _real_basis(l3)
    W = np.einsum('ai,bj,ck,ijk->abc', U1, U2, U3.conj(), C.astype(np.complex128))
    Wr, Wi = W.real, W.imag
    Wp = Wr if np.linalg.norm(Wr) >= np.linalg.norm(Wi) else Wi
    nrm = np.linalg.norm(Wp)
    if nrm > 0:
        Wp = Wp / nrm
    return Wp.astype(np.float32)


_PATHS = [(l1, l2, l3) for l1 in range(LMAX + 1) for l2 in range(LMAX + 1) for l3 in range(abs(l1 - l2), min(l1 + l2, LMAX) + 1)]
_W3J = [_w3j_real(*p) for p in _PATHS]


def _np_harm(n):
    x, y, z = n[:, 0], n[:, 1], n[:, 2]
    Y0 = np.ones((n.shape[0], 1))
    Y1 = math.sqrt(3.0) * n
    Y2 = np.stack([
        math.sqrt(15.0) * x * y,
        math.sqrt(15.0) * y * z,
        math.sqrt(5.0) / 2.0 * (3.0 * z * z - 1.0),
        math.sqrt(15.0) * x * z,
        math.sqrt(15.0) / 2.0 * (x * x - y * y)], axis=-1)
    return [Y0, Y1, Y2]


def _build_constants():
    rng = np.random.default_rng(12345)
    n = rng.normal(size=(4000, 3))
    n /= np.linalg.norm(n, axis=1, keepdims=True)
    Y = _np_harm(n)
    Z = [np.einsum('ei,ej,ijk->ek', Y[p[0]], Y[p[1]], W.astype(np.float64))
         for p, W in zip(_PATHS, _W3J)]
    cp = {}
    for p in [0, 1, 2, 3, 4, 9, 12, 14]:
        l3 = _PATHS[p][2]
        cp[p] = float((Z[p] * Y[l3]).sum() / (Y[l3] * Y[l3]).sum())
    A6, *_ = np.linalg.lstsq(Y[2], Z[6], rcond=None)
    W7 = _W3J[7].astype(np.float64).reshape(15, 3)
    W8 = _W3J[8].astype(np.float64).reshape(15, 5)
    s8 = 1.0 / math.sqrt(8.0)
    ALPHA = np.zeros((15, 6))
    for p, b in [(0, 0), (4, 0), (12, 0), (1, 1), (3, 1), (2, 3), (9, 3), (14, 3)]:
        ALPHA[p, b] = cp[p] * s8
    ALPHA[7, 2] = s8
    ALPHA[10, 2] = s8
    ALPHA[6, 4] = s8
    ALPHA[8, 5] = s8
    ALPHA[11, 5] = -s8
    WZX = np.zeros((13, 20))
    WZX[0:3, 0:15] = W7.T
    WZX[3:8, 0:15] = W8.T
    WZX[8:13, 15:20] = A6.T
    return ALPHA.astype(np.float32), WZX.astype(np.float32)


_ALPHA, _WZX = _build_constants()
_CENTERS = np.linspace(0.0, MAX_RADIUS, NUM_BASIS + 2)[1:-1].astype(np.float32).reshape(NUM_BASIS, 1)
_RSTEP = float(MAX_RADIUS / (NUM_BASIS + 1))
_EMB_SCALE = float(math.sqrt(NUM_BASIS) / 1.12)


def _silu(x):
    return x * (1.0 / (1.0 + jnp.exp(-x)))


def _prep_body(pos_ref, a_ref, emb_ref, w1_ref, b1_ref, w2_ref, b2_ref, tbl_ref):
    av = a_ref[...]
    io = lax.broadcasted_iota(jnp.int32, (N_PAD, 16), 1)
    oh = jnp.where(io == av, 1.0, 0.0).astype(jnp.float32)
    x = jnp.dot(oh, emb_ref[...], preferred_element_type=jnp.float32)
    h = _silu(jnp.dot(x, w1_ref[...], preferred_element_type=jnp.float32) + b1_ref[...])
    ai = jnp.dot(h, w2_ref[...], preferred_element_type=jnp.float32) + b2_ref[...]
    tbl_ref[...] = jnp.concatenate(
        [pos_ref[...], ai, jnp.zeros((N_PAD, 116), jnp.float32)], axis=1)


def _gather_body(tbl_hbm, src_hbm, dst_hbm, osrc_hbm, odst_hbm,
                 idx_s, idx_d, row_s, row_d, sem_s, sem_d):
    wid = lax.axis_index("s") * 2 + lax.axis_index("c")
    base = wid * (CHUNKS * CH)

    def body(i, carry):
        off = base + i * CH
        pltpu.sync_copy(src_hbm.at[pl.ds(off, CH)], idx_s)
        pltpu.sync_copy(dst_hbm.at[pl.ds(off, CH)], idx_d)
        a = pltpu.async_copy(tbl_hbm.at[idx_s], row_s, sem_s)
        b = pltpu.async_copy(tbl_hbm.at[idx_d], row_d, sem_d)
        a.wait()
        b.wait()
        pltpu.sync_copy(row_s, osrc_hbm.at[pl.ds(off, CH)])
        pltpu.sync_copy(row_d, odst_hbm.at[pl.ds(off, CH)])
        return carry

    lax.fori_loop(0, CHUNKS, body, 0)


def _edge_body(s_ref, d_ref, wzx_ref, tmt_ref, w1_ref, b1_ref, w2_ref, b2_ref,
               w3_ref, b3_ref, o_ref):
    B = EB
    St = s_ref[:, 0:16].T
    Dt = d_ref[:, 0:16].T
    v3 = Dt[0:3, :] - St[0:3, :]
    ln = jnp.sqrt(v3[0:1, :] * v3[0:1, :] + v3[1:2, :] * v3[1:2, :] + v3[2:3, :] * v3[2:3, :])
    inv = 1.0 / jnp.maximum(ln, 1e-8)
    nv = v3 * inv
    x, y, z = nv[0:1, :], nv[1:2, :], nv[2:3, :]
    s3 = math.sqrt(3.0)
    s15 = math.sqrt(15.0)
    Y1 = s3 * nv
    Y2 = jnp.concatenate([
        s15 * x * y,
        s15 * y * z,
        (math.sqrt(5.0) / 2.0) * (3.0 * z * z - 1.0),
        s15 * x * z,
        (s15 / 2.0) * (x * x - y * y)], axis=0)
    PY = (Y1[:, None, :] * Y2[None, :, :]).reshape(15, B)
    CY = jnp.concatenate([PY, Y2], axis=0)
    ZX = jnp.dot(wzx_ref[...], CY, preferred_element_type=jnp.float32)
    Z7 = ZX[0:3, :]
    Z8 = ZX[3:8, :]
    Z6 = ZX[8:13, :]

    cen = (lax.broadcasted_iota(jnp.int32, (8, 1), 0).astype(jnp.float32) + 1.0) * _RSTEP
    dif = (ln - cen) * (1.0 / _RSTEP)
    emb_t = jnp.exp(-(dif * dif)) * _EMB_SCALE
    emb = emb_t.T
    h1 = _silu(jnp.dot(emb, w1_ref[...], preferred_element_type=jnp.float32) + b1_ref[...])
    h2 = _silu(jnp.dot(h1, w2_ref[...], preferred_element_type=jnp.float32) + b2_ref[...])
    g = jnp.dot(h2, w3_ref[...], preferred_element_type=jnp.float32) + b3_ref[...]
    gt = g.T
    Ait = St[4:12, :]
    X2 = (gt[:, None, :] * Ait[None, :, :]).reshape(120, B)
    m = jnp.dot(tmt_ref[...], X2, preferred_element_type=jnp.float32)

    A1 = (m[8:16, :][:, None, :] * Y1[None, :, :]
          + m[16:24, :][:, None, :] * Z7[None, :, :]).reshape(24, B)
    A2 = (m[24:32, :][:, None, :] * Y2[None, :, :]
          + m[32:40, :][:, None, :] * Z6[None, :, :]
          + m[40:48, :][:, None, :] * Z8[None, :, :]).reshape(40, B)
    EF = jnp.concatenate([m[0:8, :], A1, A2], axis=0)
    o_ref[:, 0:72] = EF.T
    o_ref[:, 72:128] = jnp.zeros((B, 56), jnp.float32)


def _scatter_body(dst_hbm, ef_hbm, zer_hbm, out_hbm, idx_v, row_v, acc, sem):
    cid = lax.axis_index("c")
    sid = lax.axis_index("s")
    wid = sid * 2 + cid
    pltpu.sync_copy(zer_hbm.at[pl.ds(sid * ROWS_W, ROWS_W)],
                    acc.at[pl.ds(sid * ROWS_W, ROWS_W)])
    plsc.subcore_barrier()

    def body(i, carry):
        off = wid * (CHUNKS * CH) + i * CH
        pltpu.sync_copy(dst_hbm.at[pl.ds(off, CH)], idx_v)
        pltpu.sync_copy(ef_hbm.at[pl.ds(off, CH)], row_v)
        pltpu.sync_copy(row_v, acc.at[idx_v], add=True)
        return carry

    lax.fori_loop(0, CHUNKS, body, 0)
    plsc.subcore_barrier()
    pltpu.sync_copy(acc.at[pl.ds(sid * ROWS_W, ROWS_W)],
                    out_hbm.at[cid].at[pl.ds(sid * ROWS_W, ROWS_W)])


def _combine_body(p_ref, o_ref):
    a = p_ref[0, 0:N_NODES, 0:72]
    b = p_ref[1, 0:N_NODES, 0:72]
    o_ref[...] = (a + b) * (1.0 / AVG_NEIGH)


def kernel(pos, A, batch, edge_src, edge_dst, edge_shifts, cell, emb_table,
           w1, b1, w2, b2, fc_w1, fc_b1, fc_w2, fc_b2, fc_w3, fc_b3, tp_weights):
    f32 = jnp.float32
    pos_p = jnp.concatenate([pos.astype(f32), jnp.zeros((N_PAD - N_NODES, 3), f32)], axis=0)
    pos_p4 = jnp.concatenate([pos_p, jnp.zeros((N_PAD, 1), f32)], axis=1)
    a_p = jnp.concatenate([A.astype(jnp.int32), jnp.zeros((N_PAD - N_NODES,), jnp.int32)]).reshape(N_PAD, 1)
    emb_p = jnp.concatenate([emb_table.astype(f32),
                             jnp.zeros((16 - emb_table.shape[0], 16), f32)], axis=0)
    src_p = jnp.concatenate([edge_src.astype(jnp.int32),
                             jnp.zeros((E_PAD - N_EDGES,), jnp.int32)])
    dst_p = jnp.concatenate([edge_dst.astype(jnp.int32),
                             jnp.full((E_PAD - N_EDGES,), N_PAD - 1, jnp.int32)])
    tmt = jnp.einsum('pb,puv->bvpu', jnp.asarray(_ALPHA), tp_weights.astype(f32)).reshape(48, 120)
    wzx = jnp.asarray(_WZX)
    zer = jnp.zeros((N_PAD, 128), f32)

    tbl = pl.pallas_call(
        _prep_body,
        out_shape=jax.ShapeDtypeStruct((N_PAD, 128), f32),
    )(pos_p4, a_p, emb_p, w1.astype(f32), b1.reshape(1, 64).astype(f32),
      w2.astype(f32), b2.reshape(1, 8).astype(f32))

    mesh = plsc.VectorSubcoreMesh(core_axis_name="c", subcore_axis_name="s")
    gath = functools.partial(
        pl.kernel, mesh=mesh,
        out_type=[jax.ShapeDtypeStruct((E_PAD, 128), f32),
                  jax.ShapeDtypeStruct((E_PAD, 128), f32)],
        scratch_types=[pltpu.VMEM((CH,), jnp.int32), pltpu.VMEM((CH,), jnp.int32),
                       pltpu.VMEM((CH, 128), f32), pltpu.VMEM((CH, 128), f32),
                       pltpu.SemaphoreType.DMA, pltpu.SemaphoreType.DMA],
    )(_gather_body)
    srows, drows = gath(tbl, src_p, dst_p)

    nblk = E_PAD // EB
    ef = pl.pallas_call(
        _edge_body,
        grid=(nblk,),
        in_specs=[
            pl.BlockSpec((EB, 128), lambda i: (i, 0)),
            pl.BlockSpec((EB, 128), lambda i: (i, 0)),
            pl.BlockSpec((13, 20), lambda i: (0, 0)),
            pl.BlockSpec((48, 120), lambda i: (0, 0)),
            pl.BlockSpec((8, 64), lambda i: (0, 0)),
            pl.BlockSpec((1, 64), lambda i: (0, 0)),
            pl.BlockSpec((64, 64), lambda i: (0, 0)),
            pl.BlockSpec((1, 64), lambda i: (0, 0)),
            pl.BlockSpec((64, 15), lambda i: (0, 0)),
            pl.BlockSpec((1, 15), lambda i: (0, 0)),
        ],
        out_specs=pl.BlockSpec((EB, 128), lambda i: (i, 0)),
        out_shape=jax.ShapeDtypeStruct((E_PAD, 128), f32),
    )(srows, drows, wzx, tmt,
      fc_w1.astype(f32), fc_b1.reshape(1, 64).astype(f32),
      fc_w2.astype(f32), fc_b2.reshape(1, 64).astype(f32),
      fc_w3.astype(f32), fc_b3.reshape(1, 15).astype(f32))

    scat = functools.partial(
        pl.kernel, mesh=mesh,
        out_type=jax.ShapeDtypeStruct((2, N_PAD, 128), f32),
        scratch_types=[pltpu.VMEM((CH,), jnp.int32), pltpu.VMEM((CH, 128), f32),
                       pltpu.VMEM_SHARED((N_PAD, 128), f32),
                       pltpu.SemaphoreType.DMA],
    )(_scatter_body)
    partial_out = scat(dst_p, ef, zer)

    out = pl.pallas_call(
        _combine_body,
        out_shape=jax.ShapeDtypeStruct((N_NODES, 72), f32),
    )(partial_out)
    return out


# speedup vs baseline: 21.0115x; 1.0064x over previous
import functools
import math

import jax
import jax.numpy as jnp
import numpy as np
from jax import lax
from jax.experimental import pallas as pl
from jax.experimental.pallas import tpu as pltpu
from jax.experimental.pallas import tpu_sc as plsc

LMAX = 2
NUM_BASIS = 8
MAX_RADIUS = 5.0
AVG_NEIGH = 16.0
N_NODES = 10000
N_EDGES = 640000
N_PAD = 10240
E_PAD = 643072
EB = 4096
N_W = 32
CH = 128
CHUNKS = E_PAD // (N_W * CH)
ROWS_W = N_PAD // 16


def _fact(n):
    return math.factorial(n)


def _cg(j1, m1, j2, m2, j3, m3):
    if m1 + m2 != m3:
        return 0.0
    if j3 < abs(j1 - j2) or j3 > j1 + j2:
        return 0.0
    pre = math.sqrt((2 * j3 + 1) * _fact(j1 + j2 - j3) * _fact(j1 - j2 + j3) * _fact(-j1 + j2 + j3) / _fact(j1 + j2 + j3 + 1))
    pre *= math.sqrt(_fact(j3 + m3) * _fact(j3 - m3) * _fact(j1 - m1) * _fact(j1 + m1) * _fact(j2 - m2) * _fact(j2 + m2))
    kmin = max(0, j2 - j3 - m1, j1 - j3 + m2)
    kmax = min(j1 + j2 - j3, j1 - m1, j2 + m2)
    s = 0.0
    for k in range(kmin, kmax + 1):
        s += (-1.0) ** k / (_fact(k) * _fact(j1 + j2 - j3 - k) * _fact(j1 - m1 - k) * _fact(j2 + m2 - k) * _fact(j3 - j2 + m1 + k) * _fact(j3 - j1 - m2 + k))
    return pre * s


def _real_basis(l):
    U = np.zeros((2 * l + 1, 2 * l + 1), dtype=np.complex128)
    for m in range(-l, l + 1):
        if m < 0:
            U[m + l, m + l] = 1j / math.sqrt(2)
            U[m + l, -m + l] = -1j * (-1.0) ** abs(m) / math.sqrt(2)
        elif m == 0:
            U[l, l] = 1.0
        else:
            U[m + l, m + l] = (-1.0) ** m / math.sqrt(2)
            U[m + l, -m + l] = 1.0 / math.sqrt(2)
    return U


def _w3j_real(l1, l2, l3):
    C = np.zeros((2 * l1 + 1, 2 * l2 + 1, 2 * l3 + 1))
    for m1 in range(-l1, l1 + 1):
        for m2 in range(-l2, l2 + 1):
            for m3 in range(-l3, l3 + 1):
                C[m1 + l1, m2 + l2, m3 + l3] = _cg(l1, m1, l2, m2, l3, m3)
    U1, U2, U3 = _real_basis(l1), _real_basis(l2), _real_basis(l3)
    W = np.einsum('ai,bj,ck,ijk->abc', U1, U2, U3.conj(), C.astype(np.complex128))
    Wr, Wi = W.real, W.imag
    Wp = Wr if np.linalg.norm(Wr) >= np.linalg.norm(Wi) else Wi
    nrm = np.linalg.norm(Wp)
    if nrm > 0:
        Wp = Wp / nrm
    return Wp.astype(np.float32)


_PATHS = [(l1, l2, l3) for l1 in range(LMAX + 1) for l2 in range(LMAX + 1) for l3 in range(abs(l1 - l2), min(l1 + l2, LMAX) + 1)]
_W3J = [_w3j_real(*p) for p in _PATHS]


def _np_harm(n):
    x, y, z = n[:, 0], n[:, 1], n[:, 2]
    Y0 = np.ones((n.shape[0], 1))
    Y1 = math.sqrt(3.0) * n
    Y2 = np.stack([
        math.sqrt(15.0) * x * y,
        math.sqrt(15.0) * y * z,
        math.sqrt(5.0) / 2.0 * (3.0 * z * z - 1.0),
        math.sqrt(15.0) * x * z,
        math.sqrt(15.0) / 2.0 * (x * x - y * y)], axis=-1)
    return [Y0, Y1, Y2]


def _build_constants():
    rng = np.random.default_rng(12345)
    n = rng.normal(size=(4000, 3))
    n /= np.linalg.norm(n, axis=1, keepdims=True)
    Y = _np_harm(n)
    Z = [np.einsum('ei,ej,ijk->ek', Y[p[0]], Y[p[1]], W.astype(np.float64))
         for p, W in zip(_PATHS, _W3J)]
    cp = {}
    for p in [0, 1, 2, 3, 4, 9, 12, 14]:
        l3 = _PATHS[p][2]
        cp[p] = float((Z[p] * Y[l3]).sum() / (Y[l3] * Y[l3]).sum())
    A6, *_ = np.linalg.lstsq(Y[2], Z[6], rcond=None)
    W7 = _W3J[7].astype(np.float64).reshape(15, 3)
    W8 = _W3J[8].astype(np.float64).reshape(15, 5)
    s8 = 1.0 / math.sqrt(8.0)
    ALPHA = np.zeros((15, 6))
    for p, b in [(0, 0), (4, 0), (12, 0), (1, 1), (3, 1), (2, 3), (9, 3), (14, 3)]:
        ALPHA[p, b] = cp[p] * s8
    ALPHA[7, 2] = s8
    ALPHA[10, 2] = s8
    ALPHA[6, 4] = s8
    ALPHA[8, 5] = s8
    ALPHA[11, 5] = -s8
    WZX = np.zeros((13, 20))
    WZX[0:3, 0:15] = W7.T
    WZX[3:8, 0:15] = W8.T
    WZX[8:13, 15:20] = A6.T
    return ALPHA.astype(np.float32), WZX.astype(np.float32)


_ALPHA, _WZX = _build_constants()
_CENTERS = np.linspace(0.0, MAX_RADIUS, NUM_BASIS + 2)[1:-1].astype(np.float32).reshape(NUM_BASIS, 1)
_RSTEP = float(MAX_RADIUS / (NUM_BASIS + 1))
_EMB_SCALE = float(math.sqrt(NUM_BASIS) / 1.12)


def _silu(x):
    return x * (1.0 / (1.0 + jnp.exp(-x)))


def _prep_body(pos_ref, a_ref, emb_ref, w1_ref, b1_ref, w2_ref, b2_ref, tbl_ref):
    av = a_ref[...]
    io = lax.broadcasted_iota(jnp.int32, (N_PAD, 16), 1)
    oh = jnp.where(io == av, 1.0, 0.0).astype(jnp.float32)
    x = jnp.dot(oh, emb_ref[...], preferred_element_type=jnp.float32)
    h = _silu(jnp.dot(x, w1_ref[...], preferred_element_type=jnp.float32) + b1_ref[...])
    ai = jnp.dot(h, w2_ref[...], preferred_element_type=jnp.float32) + b2_ref[...]
    tbl_ref[...] = jnp.concatenate(
        [pos_ref[...], ai, jnp.zeros((N_PAD, 116), jnp.float32)], axis=1)


def _gather_body(tbl_hbm, src_hbm, dst_hbm, osrc_hbm, odst_hbm,
                 idx_s, idx_d, row_s, row_d, pk_s, pk_d, sem_s, sem_d):
    wid = lax.axis_index("s") * 2 + lax.axis_index("c")
    base = wid * (CHUNKS * CH)

    def body(i, carry):
        off = base + i * CH
        pltpu.sync_copy(src_hbm.at[pl.ds(off, CH)], idx_s)
        pltpu.sync_copy(dst_hbm.at[pl.ds(off, CH)], idx_d)
        a = pltpu.async_copy(tbl_hbm.at[idx_s], row_s, sem_s)
        b = pltpu.async_copy(tbl_hbm.at[idx_d], row_d, sem_d)
        a.wait()
        b.wait()
        for e in range(CH):
            j, k = e // 8, e % 8
            pk_s[j, 16 * k:16 * (k + 1)] = row_s[e, 0:16]
            pk_d[j, 16 * k:16 * (k + 1)] = row_d[e, 0:16]
        prow = pl.multiple_of(off // 8, 16)
        pltpu.sync_copy(pk_s, osrc_hbm.at[pl.ds(prow, CH // 8)])
        pltpu.sync_copy(pk_d, odst_hbm.at[pl.ds(prow, CH // 8)])
        return carry

    lax.fori_loop(0, CHUNKS, body, 0)


def _edge_body(s_ref, d_ref, wzx_ref, tmt_ref, w1_ref, b1_ref, w2_ref, b2_ref,
               w3_ref, b3_ref, o_ref):
    B = EB
    St = jnp.concatenate([s_ref[:, 16 * k:16 * (k + 1)].T for k in range(8)], axis=1)
    Dt = jnp.concatenate([d_ref[:, 16 * k:16 * (k + 1)].T for k in range(8)], axis=1)
    v3 = Dt[0:3, :] - St[0:3, :]
    ln = jnp.sqrt(v3[0:1, :] * v3[0:1, :] + v3[1:2, :] * v3[1:2, :] + v3[2:3, :] * v3[2:3, :])
    inv = 1.0 / jnp.maximum(ln, 1e-8)
    nv = v3 * inv
    x, y, z = nv[0:1, :], nv[1:2, :], nv[2:3, :]
    s3 = math.sqrt(3.0)
    s15 = math.sqrt(15.0)
    Y1 = s3 * nv
    Y2 = jnp.concatenate([
        s15 * x * y,
        s15 * y * z,
        (math.sqrt(5.0) / 2.0) * (3.0 * z * z - 1.0),
        s15 * x * z,
        (s15 / 2.0) * (x * x - y * y)], axis=0)
    PY = (Y1[:, None, :] * Y2[None, :, :]).reshape(15, B)
    CY = jnp.concatenate([PY, Y2], axis=0)
    ZX = jnp.dot(wzx_ref[...], CY, preferred_element_type=jnp.float32)
    Z7 = ZX[0:3, :]
    Z8 = ZX[3:8, :]
    Z6 = ZX[8:13, :]

    cen = (lax.broadcasted_iota(jnp.int32, (8, 1), 0).astype(jnp.float32) + 1.0) * _RSTEP
    dif = (ln - cen) * (1.0 / _RSTEP)
    emb_t = jnp.exp(-(dif * dif)) * _EMB_SCALE
    emb = emb_t.T
    h1 = _silu(jnp.dot(emb, w1_ref[...], preferred_element_type=jnp.float32) + b1_ref[...])
    h2 = _silu(jnp.dot(h1, w2_ref[...], preferred_element_type=jnp.float32) + b2_ref[...])
    g = jnp.dot(h2, w3_ref[...], preferred_element_type=jnp.float32) + b3_ref[...]
    gt = g.T
    Ait = St[4:12, :]
    X2 = (gt[:, None, :] * Ait[None, :, :]).reshape(120, B)
    m = jnp.dot(tmt_ref[...], X2, preferred_element_type=jnp.float32)

    A1 = (m[8:16, :][:, None, :] * Y1[None, :, :]
          + m[16:24, :][:, None, :] * Z7[None, :, :]).reshape(24, B)
    A2 = (m[24:32, :][:, None, :] * Y2[None, :, :]
          + m[32:40, :][:, None, :] * Z6[None, :, :]
          + m[40:48, :][:, None, :] * Z8[None, :, :]).reshape(40, B)
    EF = jnp.concatenate([m[0:8, :], A1, A2], axis=0)
    o_ref[:, 0:72] = EF.T
    o_ref[:, 72:128] = jnp.zeros((B, 56), jnp.float32)


def _scatter_body(dst_hbm, ef_hbm, zer_hbm, out_hbm, idx_v, row_v, acc, sem):
    cid = lax.axis_index("c")
    sid = lax.axis_index("s")
    wid = sid * 2 + cid
    pltpu.sync_copy(zer_hbm.at[pl.ds(sid * ROWS_W, ROWS_W)],
                    acc.at[pl.ds(sid * ROWS_W, ROWS_W)])
    plsc.subcore_barrier()

    def body(i, carry):
        off = wid * (CHUNKS * CH) + i * CH
        pltpu.sync_copy(dst_hbm.at[pl.ds(off, CH)], idx_v)
        pltpu.sync_copy(ef_hbm.at[pl.ds(off, CH)], row_v)
        pltpu.sync_copy(row_v, acc.at[idx_v], add=True)
        return carry

    lax.fori_loop(0, CHUNKS, body, 0)
    plsc.subcore_barrier()
    pltpu.sync_copy(acc.at[pl.ds(sid * ROWS_W, ROWS_W)],
                    out_hbm.at[cid].at[pl.ds(sid * ROWS_W, ROWS_W)])


def _combine_body(p_ref, o_ref):
    a = p_ref[0, 0:N_NODES, 0:72]
    b = p_ref[1, 0:N_NODES, 0:72]
    o_ref[...] = (a + b) * (1.0 / AVG_NEIGH)


def kernel(pos, A, batch, edge_src, edge_dst, edge_shifts, cell, emb_table,
           w1, b1, w2, b2, fc_w1, fc_b1, fc_w2, fc_b2, fc_w3, fc_b3, tp_weights):
    f32 = jnp.float32
    pos_p = jnp.concatenate([pos.astype(f32), jnp.zeros((N_PAD - N_NODES, 3), f32)], axis=0)
    pos_p4 = jnp.concatenate([pos_p, jnp.zeros((N_PAD, 1), f32)], axis=1)
    a_p = jnp.concatenate([A.astype(jnp.int32), jnp.zeros((N_PAD - N_NODES,), jnp.int32)]).reshape(N_PAD, 1)
    emb_p = jnp.concatenate([emb_table.astype(f32),
                             jnp.zeros((16 - emb_table.shape[0], 16), f32)], axis=0)
    src_p = jnp.concatenate([edge_src.astype(jnp.int32),
                             jnp.zeros((E_PAD - N_EDGES,), jnp.int32)])
    dst_p = jnp.concatenate([edge_dst.astype(jnp.int32),
                             jnp.full((E_PAD - N_EDGES,), N_PAD - 1, jnp.int32)])
    dst_perm = dst_p.reshape(E_PAD // EB, EB // CH, 16, 8).transpose(0, 3, 1, 2).reshape(-1)
    tmt = jnp.einsum('pb,puv->bvpu', jnp.asarray(_ALPHA), tp_weights.astype(f32)).reshape(48, 120)
    wzx = jnp.asarray(_WZX)
    zer = jnp.zeros((N_PAD, 128), f32)

    tbl = pl.pallas_call(
        _prep_body,
        out_shape=jax.ShapeDtypeStruct((N_PAD, 128), f32),
    )(pos_p4, a_p, emb_p, w1.astype(f32), b1.reshape(1, 64).astype(f32),
      w2.astype(f32), b2.reshape(1, 8).astype(f32))

    mesh = plsc.VectorSubcoreMesh(core_axis_name="c", subcore_axis_name="s")
    gath = functools.partial(
        pl.kernel, mesh=mesh,
        out_type=[jax.ShapeDtypeStruct((E_PAD // 8, 128), f32),
                  jax.ShapeDtypeStruct((E_PAD // 8, 128), f32)],
        scratch_types=[pltpu.VMEM((CH,), jnp.int32), pltpu.VMEM((CH,), jnp.int32),
                       pltpu.VMEM((CH, 128), f32), pltpu.VMEM((CH, 128), f32),
                       pltpu.VMEM((CH // 8, 128), f32), pltpu.VMEM((CH // 8, 128), f32),
                       pltpu.SemaphoreType.DMA, pltpu.SemaphoreType.DMA],
    )(_gather_body)
    srows, drows = gath(tbl, src_p, dst_p)

    nblk = E_PAD // EB
    ef = pl.pallas_call(
        _edge_body,
        grid=(nblk,),
        in_specs=[
            pl.BlockSpec((EB // 8, 128), lambda i: (i, 0)),
            pl.BlockSpec((EB // 8, 128), lambda i: (i, 0)),
            pl.BlockSpec((13, 20), lambda i: (0, 0)),
            pl.BlockSpec((48, 120), lambda i: (0, 0)),
            pl.BlockSpec((8, 64), lambda i: (0, 0)),
            pl.BlockSpec((1, 64), lambda i: (0, 0)),
            pl.BlockSpec((64, 64), lambda i: (0, 0)),
            pl.BlockSpec((1, 64), lambda i: (0, 0)),
            pl.BlockSpec((64, 15), lambda i: (0, 0)),
            pl.BlockSpec((1, 15), lambda i: (0, 0)),
        ],
        out_specs=pl.BlockSpec((EB, 128), lambda i: (i, 0)),
        out_shape=jax.ShapeDtypeStruct((E_PAD, 128), f32),
    )(srows, drows, wzx, tmt,
      fc_w1.astype(f32), fc_b1.reshape(1, 64).astype(f32),
      fc_w2.astype(f32), fc_b2.reshape(1, 64).astype(f32),
      fc_w3.astype(f32), fc_b3.reshape(1, 15).astype(f32))

    scat = functools.partial(
        pl.kernel, mesh=mesh,
        out_type=jax.ShapeDtypeStruct((2, N_PAD, 128), f32),
        scratch_types=[pltpu.VMEM((CH,), jnp.int32), pltpu.VMEM((CH, 128), f32),
                       pltpu.VMEM_SHARED((N_PAD, 128), f32),
                       pltpu.SemaphoreType.DMA],
    )(_scatter_body)
    partial_out = scat(dst_perm, ef, zer)

    out = pl.pallas_call(
        _combine_body,
        out_shape=jax.ShapeDtypeStruct((N_NODES, 72), f32),
    )(partial_out)
    return out


# speedup vs baseline: 33.1264x; 1.5766x over previous
import functools
import math

import jax
import jax.numpy as jnp
import numpy as np
from jax import lax
from jax.experimental import pallas as pl
from jax.experimental.pallas import tpu as pltpu
from jax.experimental.pallas import tpu_sc as plsc

LMAX = 2
NUM_BASIS = 8
MAX_RADIUS = 5.0
AVG_NEIGH = 16.0
N_NODES = 10000
N_EDGES = 640000
N_PAD = 10240
E_PAD = 643072
EB = 4096
N_W = 32
CH = 128
CHUNKS = E_PAD // (N_W * CH)
PART_BLOCKS = (40, 40, 40, 37)
ROWS_W = N_PAD // 16


def _fact(n):
    return math.factorial(n)


def _cg(j1, m1, j2, m2, j3, m3):
    if m1 + m2 != m3:
        return 0.0
    if j3 < abs(j1 - j2) or j3 > j1 + j2:
        return 0.0
    pre = math.sqrt((2 * j3 + 1) * _fact(j1 + j2 - j3) * _fact(j1 - j2 + j3) * _fact(-j1 + j2 + j3) / _fact(j1 + j2 + j3 + 1))
    pre *= math.sqrt(_fact(j3 + m3) * _fact(j3 - m3) * _fact(j1 - m1) * _fact(j1 + m1) * _fact(j2 - m2) * _fact(j2 + m2))
    kmin = max(0, j2 - j3 - m1, j1 - j3 + m2)
    kmax = min(j1 + j2 - j3, j1 - m1, j2 + m2)
    s = 0.0
    for k in range(kmin, kmax + 1):
        s += (-1.0) ** k / (_fact(k) * _fact(j1 + j2 - j3 - k) * _fact(j1 - m1 - k) * _fact(j2 + m2 - k) * _fact(j3 - j2 + m1 + k) * _fact(j3 - j1 - m2 + k))
    return pre * s


def _real_basis(l):
    U = np.zeros((2 * l + 1, 2 * l + 1), dtype=np.complex128)
    for m in range(-l, l + 1):
        if m < 0:
            U[m + l, m + l] = 1j / math.sqrt(2)
            U[m + l, -m + l] = -1j * (-1.0) ** abs(m) / math.sqrt(2)
        elif m == 0:
            U[l, l] = 1.0
        else:
            U[m + l, m + l] = (-1.0) ** m / math.sqrt(2)
            U[m + l, -m + l] = 1.0 / math.sqrt(2)
    return U


def _w3j_real(l1, l2, l3):
    C = np.zeros((2 * l1 + 1, 2 * l2 + 1, 2 * l3 + 1))
    for m1 in range(-l1, l1 + 1):
        for m2 in range(-l2, l2 + 1):
            for m3 in range(-l3, l3 + 1):
                C[m1 + l1, m2 + l2, m3 + l3] = _cg(l1, m1, l2, m2, l3, m3)
    U1, U2, U3 = _real_basis(l1), _real_basis(l2), _real_basis(l3)
    W = np.einsum('ai,bj,ck,ijk->abc', U1, U2, U3.conj(), C.astype(np.complex128))
    Wr, Wi = W.real, W.imag
    Wp = Wr if np.linalg.norm(Wr) >= np.linalg.norm(Wi) else Wi
    nrm = np.linalg.norm(Wp)
    if nrm > 0:
        Wp = Wp / nrm
    return Wp.astype(np.float32)


_PATHS = [(l1, l2, l3) for l1 in range(LMAX + 1) for l2 in range(LMAX + 1) for l3 in range(abs(l1 - l2), min(l1 + l2, LMAX) + 1)]
_W3J = [_w3j_real(*p) for p in _PATHS]


def _np_harm(n):
    x, y, z = n[:, 0], n[:, 1], n[:, 2]
    Y0 = np.ones((n.shape[0], 1))
    Y1 = math.sqrt(3.0) * n
    Y2 = np.stack([
        math.sqrt(15.0) * x * y,
        math.sqrt(15.0) * y * z,
        math.sqrt(5.0) / 2.0 * (3.0 * z * z - 1.0),
        math.sqrt(15.0) * x * z,
        math.sqrt(15.0) / 2.0 * (x * x - y * y)], axis=-1)
    return [Y0, Y1, Y2]


def _build_constants():
    rng = np.random.default_rng(12345)
    n = rng.normal(size=(4000, 3))
    n /= np.linalg.norm(n, axis=1, keepdims=True)
    Y = _np_harm(n)
    Z = [np.einsum('ei,ej,ijk->ek', Y[p[0]], Y[p[1]], W.astype(np.float64))
         for p, W in zip(_PATHS, _W3J)]
    cp = {}
    for p in [0, 1, 2, 3, 4, 9, 12, 14]:
        l3 = _PATHS[p][2]
        cp[p] = float((Z[p] * Y[l3]).sum() / (Y[l3] * Y[l3]).sum())
    A6, *_ = np.linalg.lstsq(Y[2], Z[6], rcond=None)
    W7 = _W3J[7].astype(np.float64).reshape(15, 3)
    W8 = _W3J[8].astype(np.float64).reshape(15, 5)
    s8 = 1.0 / math.sqrt(8.0)
    ALPHA = np.zeros((15, 6))
    for p, b in [(0, 0), (4, 0), (12, 0), (1, 1), (3, 1), (2, 3), (9, 3), (14, 3)]:
        ALPHA[p, b] = cp[p] * s8
    ALPHA[7, 2] = s8
    ALPHA[10, 2] = s8
    ALPHA[6, 4] = s8
    ALPHA[8, 5] = s8
    ALPHA[11, 5] = -s8
    WZX = np.zeros((13, 20))
    WZX[0:3, 0:15] = W7.T
    WZX[3:8, 0:15] = W8.T
    WZX[8:13, 15:20] = A6.T
    return ALPHA.astype(np.float32), WZX.astype(np.float32)


_ALPHA, _WZX = _build_constants()
_CENTERS = np.linspace(0.0, MAX_RADIUS, NUM_BASIS + 2)[1:-1].astype(np.float32).reshape(NUM_BASIS, 1)
_RSTEP = float(MAX_RADIUS / (NUM_BASIS + 1))
_EMB_SCALE = float(math.sqrt(NUM_BASIS) / 1.12)


def _silu(x):
    return x * (1.0 / (1.0 + jnp.exp(-x)))


def _prep_body(pos_ref, a_ref, emb_ref, w1_ref, b1_ref, w2_ref, b2_ref, tbl_ref):
    av = a_ref[...]
    io = lax.broadcasted_iota(jnp.int32, (N_PAD, 16), 1)
    oh = jnp.where(io == av, 1.0, 0.0).astype(jnp.float32)
    x = jnp.dot(oh, emb_ref[...], preferred_element_type=jnp.float32)
    h = _silu(jnp.dot(x, w1_ref[...], preferred_element_type=jnp.float32) + b1_ref[...])
    ai = jnp.dot(h, w2_ref[...], preferred_element_type=jnp.float32) + b2_ref[...]
    tbl_ref[...] = jnp.concatenate(
        [pos_ref[...], ai, jnp.zeros((N_PAD, 116), jnp.float32)], axis=1)


def _make_gather(nchunks):
    def _gather_body(tbl_hbm, src_hbm, dst_hbm, osrc_hbm, odst_hbm,
                     idx_s, idx_d, row_s, row_d, pk_s, pk_d, sem_s, sem_d):
        wid = lax.axis_index("s") * 2 + lax.axis_index("c")
        base = wid * (nchunks * CH)

        def body(i, carry):
            off = base + i * CH
            pltpu.sync_copy(src_hbm.at[pl.ds(off, CH)], idx_s)
            pltpu.sync_copy(dst_hbm.at[pl.ds(off, CH)], idx_d)
            a = pltpu.async_copy(tbl_hbm.at[idx_s], row_s, sem_s)
            b = pltpu.async_copy(tbl_hbm.at[idx_d], row_d, sem_d)
            a.wait()
            b.wait()
            for e in range(CH):
                j, k = e // 8, e % 8
                pk_s[j, 16 * k:16 * (k + 1)] = row_s[e, 0:16]
                pk_d[j, 16 * k:16 * (k + 1)] = row_d[e, 0:16]
            prow = pl.multiple_of(off // 8, 16)
            pltpu.sync_copy(pk_s, osrc_hbm.at[pl.ds(prow, CH // 8)])
            pltpu.sync_copy(pk_d, odst_hbm.at[pl.ds(prow, CH // 8)])
            return carry

        lax.fori_loop(0, nchunks, body, 0)

    return _gather_body


def _edge_body(s_ref, d_ref, wzx_ref, tmt_ref, w1_ref, b1_ref, w2_ref, b2_ref,
               w3_ref, b3_ref, o_ref):
    B = EB
    St = jnp.concatenate([s_ref[:, 16 * k:16 * (k + 1)].T for k in range(8)], axis=1)
    Dt = jnp.concatenate([d_ref[:, 16 * k:16 * (k + 1)].T for k in range(8)], axis=1)
    v3 = Dt[0:3, :] - St[0:3, :]
    ln = jnp.sqrt(v3[0:1, :] * v3[0:1, :] + v3[1:2, :] * v3[1:2, :] + v3[2:3, :] * v3[2:3, :])
    inv = 1.0 / jnp.maximum(ln, 1e-8)
    nv = v3 * inv
    x, y, z = nv[0:1, :], nv[1:2, :], nv[2:3, :]
    s3 = math.sqrt(3.0)
    s15 = math.sqrt(15.0)
    Y1 = s3 * nv
    Y2 = jnp.concatenate([
        s15 * x * y,
        s15 * y * z,
        (math.sqrt(5.0) / 2.0) * (3.0 * z * z - 1.0),
        s15 * x * z,
        (s15 / 2.0) * (x * x - y * y)], axis=0)
    PY = (Y1[:, None, :] * Y2[None, :, :]).reshape(15, B)
    CY = jnp.concatenate([PY, Y2], axis=0)
    ZX = jnp.dot(wzx_ref[...], CY, preferred_element_type=jnp.float32)
    Z7 = ZX[0:3, :]
    Z8 = ZX[3:8, :]
    Z6 = ZX[8:13, :]

    cen = (lax.broadcasted_iota(jnp.int32, (8, 1), 0).astype(jnp.float32) + 1.0) * _RSTEP
    dif = (ln - cen) * (1.0 / _RSTEP)
    emb_t = jnp.exp(-(dif * dif)) * _EMB_SCALE
    emb = emb_t.T
    h1 = _silu(jnp.dot(emb, w1_ref[...], preferred_element_type=jnp.float32) + b1_ref[...])
    h2 = _silu(jnp.dot(h1, w2_ref[...], preferred_element_type=jnp.float32) + b2_ref[...])
    g = jnp.dot(h2, w3_ref[...], preferred_element_type=jnp.float32) + b3_ref[...]
    gt = g.T
    Ait = St[4:12, :]
    X2 = (gt[:, None, :] * Ait[None, :, :]).reshape(120, B)
    m = jnp.dot(tmt_ref[...], X2, preferred_element_type=jnp.float32)

    A1 = (m[8:16, :][:, None, :] * Y1[None, :, :]
          + m[16:24, :][:, None, :] * Z7[None, :, :]).reshape(24, B)
    A2 = (m[24:32, :][:, None, :] * Y2[None, :, :]
          + m[32:40, :][:, None, :] * Z6[None, :, :]
          + m[40:48, :][:, None, :] * Z8[None, :, :]).reshape(40, B)
    EF = jnp.concatenate([m[0:8, :], A1, A2], axis=0)
    o_ref[:, 0:72] = EF.T
    o_ref[:, 72:128] = jnp.zeros((B, 56), jnp.float32)


def _make_scatter(nchunks):
    def _scatter_body(dst_hbm, ef_hbm, zer_hbm, out_hbm, idx_v, row_v, acc, sem):
        cid = lax.axis_index("c")
        sid = lax.axis_index("s")
        wid = sid * 2 + cid
        pltpu.sync_copy(zer_hbm.at[pl.ds(sid * ROWS_W, ROWS_W)],
                        acc.at[pl.ds(sid * ROWS_W, ROWS_W)])
        plsc.subcore_barrier()

        def body(i, carry):
            off = wid * (nchunks * CH) + i * CH
            pltpu.sync_copy(dst_hbm.at[pl.ds(off, CH)], idx_v)
            pltpu.sync_copy(ef_hbm.at[pl.ds(off, CH)], row_v)
            pltpu.sync_copy(row_v, acc.at[idx_v], add=True)
            return carry

        lax.fori_loop(0, nchunks, body, 0)
        plsc.subcore_barrier()
        pltpu.sync_copy(acc.at[pl.ds(sid * ROWS_W, ROWS_W)],
                        out_hbm.at[cid].at[pl.ds(sid * ROWS_W, ROWS_W)])

    return _scatter_body


def _combine_body(*refs):
    parts, o_ref = refs[:-1], refs[-1]
    s = parts[0][0, 0:N_NODES, 0:72] + parts[0][1, 0:N_NODES, 0:72]
    for p_ref in parts[1:]:
        s = s + p_ref[0, 0:N_NODES, 0:72] + p_ref[1, 0:N_NODES, 0:72]
    o_ref[...] = s * (1.0 / AVG_NEIGH)


def kernel(pos, A, batch, edge_src, edge_dst, edge_shifts, cell, emb_table,
           w1, b1, w2, b2, fc_w1, fc_b1, fc_w2, fc_b2, fc_w3, fc_b3, tp_weights):
    f32 = jnp.float32
    pos_p = jnp.concatenate([pos.astype(f32), jnp.zeros((N_PAD - N_NODES, 3), f32)], axis=0)
    pos_p4 = jnp.concatenate([pos_p, jnp.zeros((N_PAD, 1), f32)], axis=1)
    a_p = jnp.concatenate([A.astype(jnp.int32), jnp.zeros((N_PAD - N_NODES,), jnp.int32)]).reshape(N_PAD, 1)
    emb_p = jnp.concatenate([emb_table.astype(f32),
                             jnp.zeros((16 - emb_table.shape[0], 16), f32)], axis=0)
    src_p = jnp.concatenate([edge_src.astype(jnp.int32),
                             jnp.zeros((E_PAD - N_EDGES,), jnp.int32)])
    dst_p = jnp.concatenate([edge_dst.astype(jnp.int32),
                             jnp.full((E_PAD - N_EDGES,), N_PAD - 1, jnp.int32)])
    dst_perm = dst_p.reshape(E_PAD // EB, EB // CH, 16, 8).transpose(0, 3, 1, 2).reshape(-1)
    tmt = jnp.einsum('pb,puv->bvpu', jnp.asarray(_ALPHA), tp_weights.astype(f32)).reshape(48, 120)
    wzx = jnp.asarray(_WZX)
    zer = jnp.zeros((N_PAD, 128), f32)

    tbl = pl.pallas_call(
        _prep_body,
        out_shape=jax.ShapeDtypeStruct((N_PAD, 128), f32),
    )(pos_p4, a_p, emb_p, w1.astype(f32), b1.reshape(1, 64).astype(f32),
      w2.astype(f32), b2.reshape(1, 8).astype(f32))

    mesh = plsc.VectorSubcoreMesh(core_axis_name="c", subcore_axis_name="s")
    partials = []
    b0 = 0
    for nb in PART_BLOCKS:
        ne = nb * EB
        e0 = b0 * EB
        gath = functools.partial(
            pl.kernel, mesh=mesh,
            out_type=[jax.ShapeDtypeStruct((ne // 8, 128), f32),
                      jax.ShapeDtypeStruct((ne // 8, 128), f32)],
            scratch_types=[pltpu.VMEM((CH,), jnp.int32), pltpu.VMEM((CH,), jnp.int32),
                           pltpu.VMEM((CH, 128), f32), pltpu.VMEM((CH, 128), f32),
                           pltpu.VMEM((CH // 8, 128), f32), pltpu.VMEM((CH // 8, 128), f32),
                           pltpu.SemaphoreType.DMA, pltpu.SemaphoreType.DMA],
        )(_make_gather(nb))
        srows, drows = gath(tbl, lax.dynamic_slice(src_p, (e0,), (ne,)),
                            lax.dynamic_slice(dst_p, (e0,), (ne,)))

        ef = pl.pallas_call(
            _edge_body,
            grid=(nb,),
            in_specs=[
                pl.BlockSpec((EB // 8, 128), lambda i: (i, 0)),
                pl.BlockSpec((EB // 8, 128), lambda i: (i, 0)),
                pl.BlockSpec((13, 20), lambda i: (0, 0)),
                pl.BlockSpec((48, 120), lambda i: (0, 0)),
                pl.BlockSpec((8, 64), lambda i: (0, 0)),
                pl.BlockSpec((1, 64), lambda i: (0, 0)),
                pl.BlockSpec((64, 64), lambda i: (0, 0)),
                pl.BlockSpec((1, 64), lambda i: (0, 0)),
                pl.BlockSpec((64, 15), lambda i: (0, 0)),
                pl.BlockSpec((1, 15), lambda i: (0, 0)),
            ],
            out_specs=pl.BlockSpec((EB, 128), lambda i: (i, 0)),
            out_shape=jax.ShapeDtypeStruct((ne, 128), f32),
        )(srows, drows, wzx, tmt,
          fc_w1.astype(f32), fc_b1.reshape(1, 64).astype(f32),
          fc_w2.astype(f32), fc_b2.reshape(1, 64).astype(f32),
          fc_w3.astype(f32), fc_b3.reshape(1, 15).astype(f32))

        scat = functools.partial(
            pl.kernel, mesh=mesh,
            out_type=jax.ShapeDtypeStruct((2, N_PAD, 128), f32),
            scratch_types=[pltpu.VMEM((CH,), jnp.int32), pltpu.VMEM((CH, 128), f32),
                           pltpu.VMEM_SHARED((N_PAD, 128), f32),
                           pltpu.SemaphoreType.DMA],
        )(_make_scatter(nb))
        partials.append(scat(lax.dynamic_slice(dst_perm, (e0,), (ne,)), ef, zer))
        b0 += nb

    out = pl.pallas_call(
        _combine_body,
        out_shape=jax.ShapeDtypeStruct((N_NODES, 72), f32),
    )(*partials)
    return out


# speedup vs baseline: 36.5233x; 1.1025x over previous
import functools
import math

import jax
import jax.numpy as jnp
import numpy as np
from jax import lax
from jax.experimental import pallas as pl
from jax.experimental.pallas import tpu as pltpu
from jax.experimental.pallas import tpu_sc as plsc

LMAX = 2
NUM_BASIS = 8
MAX_RADIUS = 5.0
AVG_NEIGH = 16.0
N_NODES = 10000
N_EDGES = 640000
N_PAD = 10240
E_PAD = 643072
EB = 4096
N_W = 32
CH = 128
CHUNKS = E_PAD // (N_W * CH)
PART_BLOCKS = (40, 40, 40, 37)
ROWS_W = N_PAD // 16


def _fact(n):
    return math.factorial(n)


def _cg(j1, m1, j2, m2, j3, m3):
    if m1 + m2 != m3:
        return 0.0
    if j3 < abs(j1 - j2) or j3 > j1 + j2:
        return 0.0
    pre = math.sqrt((2 * j3 + 1) * _fact(j1 + j2 - j3) * _fact(j1 - j2 + j3) * _fact(-j1 + j2 + j3) / _fact(j1 + j2 + j3 + 1))
    pre *= math.sqrt(_fact(j3 + m3) * _fact(j3 - m3) * _fact(j1 - m1) * _fact(j1 + m1) * _fact(j2 - m2) * _fact(j2 + m2))
    kmin = max(0, j2 - j3 - m1, j1 - j3 + m2)
    kmax = min(j1 + j2 - j3, j1 - m1, j2 + m2)
    s = 0.0
    for k in range(kmin, kmax + 1):
        s += (-1.0) ** k / (_fact(k) * _fact(j1 + j2 - j3 - k) * _fact(j1 - m1 - k) * _fact(j2 + m2 - k) * _fact(j3 - j2 + m1 + k) * _fact(j3 - j1 - m2 + k))
    return pre * s


def _real_basis(l):
    U = np.zeros((2 * l + 1, 2 * l + 1), dtype=np.complex128)
    for m in range(-l, l + 1):
        if m < 0:
            U[m + l, m + l] = 1j / math.sqrt(2)
            U[m + l, -m + l] = -1j * (-1.0) ** abs(m) / math.sqrt(2)
        elif m == 0:
            U[l, l] = 1.0
        else:
            U[m + l, m + l] = (-1.0) ** m / math.sqrt(2)
            U[m + l, -m + l] = 1.0 / math.sqrt(2)
    return U


def _w3j_real(l1, l2, l3):
    C = np.zeros((2 * l1 + 1, 2 * l2 + 1, 2 * l3 + 1))
    for m1 in range(-l1, l1 + 1):
        for m2 in range(-l2, l2 + 1):
            for m3 in range(-l3, l3 + 1):
                C[m1 + l1, m2 + l2, m3 + l3] = _cg(l1, m1, l2, m2, l3, m3)
    U1, U2, U3 = _real_basis(l1), _real_basis(l2), _real_basis(l3)
    W = np.einsum('ai,bj,ck,ijk->abc', U1, U2, U3.conj(), C.astype(np.complex128))
    Wr, Wi = W.real, W.imag
    Wp = Wr if np.linalg.norm(Wr) >= np.linalg.norm(Wi) else Wi
    nrm = np.linalg.norm(Wp)
    if nrm > 0:
        Wp = Wp / nrm
    return Wp.astype(np.float32)


_PATHS = [(l1, l2, l3) for l1 in range(LMAX + 1) for l2 in range(LMAX + 1) for l3 in range(abs(l1 - l2), min(l1 + l2, LMAX) + 1)]
_W3J = [_w3j_real(*p) for p in _PATHS]


def _np_harm(n):
    x, y, z = n[:, 0], n[:, 1], n[:, 2]
    Y0 = np.ones((n.shape[0], 1))
    Y1 = math.sqrt(3.0) * n
    Y2 = np.stack([
        math.sqrt(15.0) * x * y,
        math.sqrt(15.0) * y * z,
        math.sqrt(5.0) / 2.0 * (3.0 * z * z - 1.0),
        math.sqrt(15.0) * x * z,
        math.sqrt(15.0) / 2.0 * (x * x - y * y)], axis=-1)
    return [Y0, Y1, Y2]


def _build_constants():
    rng = np.random.default_rng(12345)
    n = rng.normal(size=(4000, 3))
    n /= np.linalg.norm(n, axis=1, keepdims=True)
    Y = _np_harm(n)
    Z = [np.einsum('ei,ej,ijk->ek', Y[p[0]], Y[p[1]], W.astype(np.float64))
         for p, W in zip(_PATHS, _W3J)]
    cp = {}
    for p in [0, 1, 2, 3, 4, 9, 12, 14]:
        l3 = _PATHS[p][2]
        cp[p] = float((Z[p] * Y[l3]).sum() / (Y[l3] * Y[l3]).sum())
    A6, *_ = np.linalg.lstsq(Y[2], Z[6], rcond=None)
    W7 = _W3J[7].astype(np.float64).reshape(15, 3)
    W8 = _W3J[8].astype(np.float64).reshape(15, 5)
    s8 = 1.0 / math.sqrt(8.0)
    ALPHA = np.zeros((15, 6))
    for p, b in [(0, 0), (4, 0), (12, 0), (1, 1), (3, 1), (2, 3), (9, 3), (14, 3)]:
        ALPHA[p, b] = cp[p] * s8
    ALPHA[7, 2] = s8
    ALPHA[10, 2] = s8
    ALPHA[6, 4] = s8
    ALPHA[8, 5] = s8
    ALPHA[11, 5] = -s8
    WZX = np.zeros((13, 20))
    WZX[0:3, 0:15] = W7.T
    WZX[3:8, 0:15] = W8.T
    WZX[8:13, 15:20] = A6.T
    return ALPHA.astype(np.float32), WZX.astype(np.float32)


_ALPHA, _WZX = _build_constants()
_CENTERS = np.linspace(0.0, MAX_RADIUS, NUM_BASIS + 2)[1:-1].astype(np.float32).reshape(NUM_BASIS, 1)
_RSTEP = float(MAX_RADIUS / (NUM_BASIS + 1))
_EMB_SCALE = float(math.sqrt(NUM_BASIS) / 1.12)


def _silu(x):
    return x * (1.0 / (1.0 + jnp.exp(-x)))


def _prep_body(pos_ref, a_ref, emb_ref, w1_ref, b1_ref, w2_ref, b2_ref, tbl_ref):
    av = a_ref[...]
    io = lax.broadcasted_iota(jnp.int32, (N_PAD, 16), 1)
    oh = jnp.where(io == av, 1.0, 0.0).astype(jnp.float32)
    x = jnp.dot(oh, emb_ref[...], preferred_element_type=jnp.float32)
    h = _silu(jnp.dot(x, w1_ref[...], preferred_element_type=jnp.float32) + b1_ref[...])
    ai = jnp.dot(h, w2_ref[...], preferred_element_type=jnp.float32) + b2_ref[...]
    tbl_ref[...] = jnp.concatenate(
        [pos_ref[...], ai, jnp.zeros((N_PAD, 116), jnp.float32)], axis=1)


def _make_gather(nchunks):
    def _gather_body(tbl_hbm, src_hbm, dst_hbm, osrc_hbm, odst_hbm,
                     idx_s0, idx_d0, row_s0, row_d0, pk_s0, pk_d0,
                     idx_s1, idx_d1, row_s1, row_d1, pk_s1, pk_d1,
                     sem_s0, sem_d0, sem_s1, sem_d1):
        wid = lax.axis_index("s") * 2 + lax.axis_index("c")
        base = wid * (nchunks * CH)
        bufs = ((idx_s0, idx_d0, row_s0, row_d0, pk_s0, pk_d0, sem_s0, sem_d0),
                (idx_s1, idx_d1, row_s1, row_d1, pk_s1, pk_d1, sem_s1, sem_d1))

        def start(i, bs):
            off = base + i * CH
            pltpu.sync_copy(src_hbm.at[pl.ds(off, CH)], bs[0])
            pltpu.sync_copy(dst_hbm.at[pl.ds(off, CH)], bs[1])
            pltpu.async_copy(tbl_hbm.at[bs[0]], bs[2], bs[6])
            pltpu.async_copy(tbl_hbm.at[bs[1]], bs[3], bs[7])

        def finish(i, bs):
            pltpu.make_async_copy(tbl_hbm.at[bs[0]], bs[2], bs[6]).wait()
            pltpu.make_async_copy(tbl_hbm.at[bs[1]], bs[3], bs[7]).wait()
            for e in range(CH):
                j, k = e // 8, e % 8
                bs[4][j, 16 * k:16 * (k + 1)] = bs[2][e, 0:16]
                bs[5][j, 16 * k:16 * (k + 1)] = bs[3][e, 0:16]
            prow = pl.multiple_of((base + i * CH) // 8, 16)
            pltpu.sync_copy(bs[4], osrc_hbm.at[pl.ds(prow, CH // 8)])
            pltpu.sync_copy(bs[5], odst_hbm.at[pl.ds(prow, CH // 8)])

        start(0, bufs[0])

        def body(i2, carry):
            i = i2 * 2

            @pl.when(i + 1 < nchunks)
            def _():
                start(i + 1, bufs[1])

            finish(i, bufs[0])

            @pl.when(i + 2 < nchunks)
            def _():
                start(i + 2, bufs[0])

            @pl.when(i + 1 < nchunks)
            def _():
                finish(i + 1, bufs[1])

            return carry

        lax.fori_loop(0, (nchunks + 1) // 2, body, 0)

    return _gather_body


def _edge_body(s_ref, d_ref, wzx_ref, tmt_ref, w1_ref, b1_ref, w2_ref, b2_ref,
               w3_ref, b3_ref, o_ref):
    B = EB
    St = jnp.concatenate([s_ref[:, 16 * k:16 * (k + 1)].T for k in range(8)], axis=1)
    Dt = jnp.concatenate([d_ref[:, 16 * k:16 * (k + 1)].T for k in range(8)], axis=1)
    v3 = Dt[0:3, :] - St[0:3, :]
    ln = jnp.sqrt(v3[0:1, :] * v3[0:1, :] + v3[1:2, :] * v3[1:2, :] + v3[2:3, :] * v3[2:3, :])
    inv = 1.0 / jnp.maximum(ln, 1e-8)
    nv = v3 * inv
    x, y, z = nv[0:1, :], nv[1:2, :], nv[2:3, :]
    s3 = math.sqrt(3.0)
    s15 = math.sqrt(15.0)
    Y1 = s3 * nv
    Y2 = jnp.concatenate([
        s15 * x * y,
        s15 * y * z,
        (math.sqrt(5.0) / 2.0) * (3.0 * z * z - 1.0),
        s15 * x * z,
        (s15 / 2.0) * (x * x - y * y)], axis=0)
    PY = (Y1[:, None, :] * Y2[None, :, :]).reshape(15, B)
    CY = jnp.concatenate([PY, Y2], axis=0)
    ZX = jnp.dot(wzx_ref[...], CY, preferred_element_type=jnp.float32)
    Z7 = ZX[0:3, :]
    Z8 = ZX[3:8, :]
    Z6 = ZX[8:13, :]

    cen = (lax.broadcasted_iota(jnp.int32, (8, 1), 0).astype(jnp.float32) + 1.0) * _RSTEP
    dif = (ln - cen) * (1.0 / _RSTEP)
    emb_t = jnp.exp(-(dif * dif)) * _EMB_SCALE
    emb = emb_t.T
    h1 = _silu(jnp.dot(emb, w1_ref[...], preferred_element_type=jnp.float32) + b1_ref[...])
    h2 = _silu(jnp.dot(h1, w2_ref[...], preferred_element_type=jnp.float32) + b2_ref[...])
    g = jnp.dot(h2, w3_ref[...], preferred_element_type=jnp.float32) + b3_ref[...]
    gt = g.T
    Ait = St[4:12, :]
    X2 = (gt[:, None, :] * Ait[None, :, :]).reshape(120, B)
    m = jnp.dot(tmt_ref[...], X2, preferred_element_type=jnp.float32)

    A1 = (m[8:16, :][:, None, :] * Y1[None, :, :]
          + m[16:24, :][:, None, :] * Z7[None, :, :]).reshape(24, B)
    A2 = (m[24:32, :][:, None, :] * Y2[None, :, :]
          + m[32:40, :][:, None, :] * Z6[None, :, :]
          + m[40:48, :][:, None, :] * Z8[None, :, :]).reshape(40, B)
    EF = jnp.concatenate([m[0:8, :], A1, A2], axis=0)
    o_ref[:, 0:72] = EF.T
    o_ref[:, 72:128] = jnp.zeros((B, 56), jnp.float32)


def _make_scatter(nchunks):
    def _scatter_body(dst_hbm, ef_hbm, zer_hbm, out_hbm,
                      idx_v0, row_v0, idx_v1, row_v1, acc, sem0, sem1):
        cid = lax.axis_index("c")
        sid = lax.axis_index("s")
        wid = sid * 2 + cid
        base = wid * (nchunks * CH)
        bufs = ((idx_v0, row_v0, sem0), (idx_v1, row_v1, sem1))
        pltpu.sync_copy(zer_hbm.at[pl.ds(sid * ROWS_W, ROWS_W)],
                        acc.at[pl.ds(sid * ROWS_W, ROWS_W)])
        plsc.subcore_barrier()

        def start(i, bs):
            off = base + i * CH
            pltpu.sync_copy(dst_hbm.at[pl.ds(off, CH)], bs[0])
            pltpu.async_copy(ef_hbm.at[pl.ds(off, CH)], bs[1], bs[2])

        def finish(i, bs):
            off = base + i * CH
            pltpu.make_async_copy(ef_hbm.at[pl.ds(off, CH)], bs[1], bs[2]).wait()
            pltpu.sync_copy(bs[1], acc.at[bs[0]], add=True)

        start(0, bufs[0])

        def body(i2, carry):
            i = i2 * 2

            @pl.when(i + 1 < nchunks)
            def _():
                start(i + 1, bufs[1])

            finish(i, bufs[0])

            @pl.when(i + 2 < nchunks)
            def _():
                start(i + 2, bufs[0])

            @pl.when(i + 1 < nchunks)
            def _():
                finish(i + 1, bufs[1])

            return carry

        lax.fori_loop(0, (nchunks + 1) // 2, body, 0)
        plsc.subcore_barrier()
        pltpu.sync_copy(acc.at[pl.ds(sid * ROWS_W, ROWS_W)],
                        out_hbm.at[cid].at[pl.ds(sid * ROWS_W, ROWS_W)])

    return _scatter_body


def _combine_body(*refs):
    parts, o_ref = refs[:-1], refs[-1]
    s = parts[0][0, 0:N_NODES, 0:72] + parts[0][1, 0:N_NODES, 0:72]
    for p_ref in parts[1:]:
        s = s + p_ref[0, 0:N_NODES, 0:72] + p_ref[1, 0:N_NODES, 0:72]
    o_ref[...] = s * (1.0 / AVG_NEIGH)


def kernel(pos, A, batch, edge_src, edge_dst, edge_shifts, cell, emb_table,
           w1, b1, w2, b2, fc_w1, fc_b1, fc_w2, fc_b2, fc_w3, fc_b3, tp_weights):
    f32 = jnp.float32
    pos_p = jnp.concatenate([pos.astype(f32), jnp.zeros((N_PAD - N_NODES, 3), f32)], axis=0)
    pos_p4 = jnp.concatenate([pos_p, jnp.zeros((N_PAD, 1), f32)], axis=1)
    a_p = jnp.concatenate([A.astype(jnp.int32), jnp.zeros((N_PAD - N_NODES,), jnp.int32)]).reshape(N_PAD, 1)
    emb_p = jnp.concatenate([emb_table.astype(f32),
                             jnp.zeros((16 - emb_table.shape[0], 16), f32)], axis=0)
    src_p = jnp.concatenate([edge_src.astype(jnp.int32),
                             jnp.zeros((E_PAD - N_EDGES,), jnp.int32)])
    dst_p = jnp.concatenate([edge_dst.astype(jnp.int32),
                             jnp.full((E_PAD - N_EDGES,), N_PAD - 1, jnp.int32)])
    dst_perm = dst_p.reshape(E_PAD // EB, EB // CH, 16, 8).transpose(0, 3, 1, 2).reshape(-1)
    tmt = jnp.einsum('pb,puv->bvpu', jnp.asarray(_ALPHA), tp_weights.astype(f32)).reshape(48, 120)
    wzx = jnp.asarray(_WZX)
    zer = jnp.zeros((N_PAD, 128), f32)

    tbl = pl.pallas_call(
        _prep_body,
        out_shape=jax.ShapeDtypeStruct((N_PAD, 128), f32),
    )(pos_p4, a_p, emb_p, w1.astype(f32), b1.reshape(1, 64).astype(f32),
      w2.astype(f32), b2.reshape(1, 8).astype(f32))

    mesh = plsc.VectorSubcoreMesh(core_axis_name="c", subcore_axis_name="s")
    partials = []
    b0 = 0
    for nb in PART_BLOCKS:
        ne = nb * EB
        e0 = b0 * EB
        gath = functools.partial(
            pl.kernel, mesh=mesh,
            out_type=[jax.ShapeDtypeStruct((ne // 8, 128), f32),
                      jax.ShapeDtypeStruct((ne // 8, 128), f32)],
            scratch_types=[pltpu.VMEM((CH,), jnp.int32), pltpu.VMEM((CH,), jnp.int32),
                           pltpu.VMEM((CH, 128), f32), pltpu.VMEM((CH, 128), f32),
                           pltpu.VMEM((CH // 8, 128), f32), pltpu.VMEM((CH // 8, 128), f32),
                           pltpu.VMEM((CH,), jnp.int32), pltpu.VMEM((CH,), jnp.int32),
                           pltpu.VMEM((CH, 128), f32), pltpu.VMEM((CH, 128), f32),
                           pltpu.VMEM((CH // 8, 128), f32), pltpu.VMEM((CH // 8, 128), f32),
                           pltpu.SemaphoreType.DMA, pltpu.SemaphoreType.DMA,
                           pltpu.SemaphoreType.DMA, pltpu.SemaphoreType.DMA],
        )(_make_gather(nb))
        srows, drows = gath(tbl, lax.dynamic_slice(src_p, (e0,), (ne,)),
                            lax.dynamic_slice(dst_p, (e0,), (ne,)))

        ef = pl.pallas_call(
            _edge_body,
            grid=(nb,),
            in_specs=[
                pl.BlockSpec((EB // 8, 128), lambda i: (i, 0)),
                pl.BlockSpec((EB // 8, 128), lambda i: (i, 0)),
                pl.BlockSpec((13, 20), lambda i: (0, 0)),
                pl.BlockSpec((48, 120), lambda i: (0, 0)),
                pl.BlockSpec((8, 64), lambda i: (0, 0)),
                pl.BlockSpec((1, 64), lambda i: (0, 0)),
                pl.BlockSpec((64, 64), lambda i: (0, 0)),
                pl.BlockSpec((1, 64), lambda i: (0, 0)),
                pl.BlockSpec((64, 15), lambda i: (0, 0)),
                pl.BlockSpec((1, 15), lambda i: (0, 0)),
            ],
            out_specs=pl.BlockSpec((EB, 128), lambda i: (i, 0)),
            out_shape=jax.ShapeDtypeStruct((ne, 128), f32),
        )(srows, drows, wzx, tmt,
          fc_w1.astype(f32), fc_b1.reshape(1, 64).astype(f32),
          fc_w2.astype(f32), fc_b2.reshape(1, 64).astype(f32),
          fc_w3.astype(f32), fc_b3.reshape(1, 15).astype(f32))

        scat = functools.partial(
            pl.kernel, mesh=mesh,
            out_type=jax.ShapeDtypeStruct((2, N_PAD, 128), f32),
            scratch_types=[pltpu.VMEM((CH,), jnp.int32), pltpu.VMEM((CH, 128), f32),
                           pltpu.VMEM((CH,), jnp.int32), pltpu.VMEM((CH, 128), f32),
                           pltpu.VMEM_SHARED((N_PAD, 128), f32),
                           pltpu.SemaphoreType.DMA, pltpu.SemaphoreType.DMA],
        )(_make_scatter(nb))
        partials.append(scat(lax.dynamic_slice(dst_perm, (e0,), (ne,)), ef, zer))
        b0 += nb

    out = pl.pallas_call(
        _combine_body,
        out_shape=jax.ShapeDtypeStruct((N_NODES, 72), f32),
    )(*partials)
    return out


# speedup vs baseline: 37.8778x; 1.0371x over previous
import functools
import math

import jax
import jax.numpy as jnp
import numpy as np
from jax import lax
from jax.experimental import pallas as pl
from jax.experimental.pallas import tpu as pltpu
from jax.experimental.pallas import tpu_sc as plsc

LMAX = 2
NUM_BASIS = 8
MAX_RADIUS = 5.0
AVG_NEIGH = 16.0
N_NODES = 10000
N_EDGES = 640000
N_PAD = 10240
E_PAD = 643072
EB = 4096
N_W = 32
CH = 128
CHUNKS = E_PAD // (N_W * CH)
PART_BLOCKS = (27, 26, 26, 26, 26, 26)
ROWS_W = N_PAD // 16


def _fact(n):
    return math.factorial(n)


def _cg(j1, m1, j2, m2, j3, m3):
    if m1 + m2 != m3:
        return 0.0
    if j3 < abs(j1 - j2) or j3 > j1 + j2:
        return 0.0
    pre = math.sqrt((2 * j3 + 1) * _fact(j1 + j2 - j3) * _fact(j1 - j2 + j3) * _fact(-j1 + j2 + j3) / _fact(j1 + j2 + j3 + 1))
    pre *= math.sqrt(_fact(j3 + m3) * _fact(j3 - m3) * _fact(j1 - m1) * _fact(j1 + m1) * _fact(j2 - m2) * _fact(j2 + m2))
    kmin = max(0, j2 - j3 - m1, j1 - j3 + m2)
    kmax = min(j1 + j2 - j3, j1 - m1, j2 + m2)
    s = 0.0
    for k in range(kmin, kmax + 1):
        s += (-1.0) ** k / (_fact(k) * _fact(j1 + j2 - j3 - k) * _fact(j1 - m1 - k) * _fact(j2 + m2 - k) * _fact(j3 - j2 + m1 + k) * _fact(j3 - j1 - m2 + k))
    return pre * s


def _real_basis(l):
    U = np.zeros((2 * l + 1, 2 * l + 1), dtype=np.complex128)
    for m in range(-l, l + 1):
        if m < 0:
            U[m + l, m + l] = 1j / math.sqrt(2)
            U[m + l, -m + l] = -1j * (-1.0) ** abs(m) / math.sqrt(2)
        elif m == 0:
            U[l, l] = 1.0
        else:
            U[m + l, m + l] = (-1.0) ** m / math.sqrt(2)
            U[m + l, -m + l] = 1.0 / math.sqrt(2)
    return U


def _w3j_real(l1, l2, l3):
    C = np.zeros((2 * l1 + 1, 2 * l2 + 1, 2 * l3 + 1))
    for m1 in range(-l1, l1 + 1):
        for m2 in range(-l2, l2 + 1):
            for m3 in range(-l3, l3 + 1):
                C[m1 + l1, m2 + l2, m3 + l3] = _cg(l1, m1, l2, m2, l3, m3)
    U1, U2, U3 = _real_basis(l1), _real_basis(l2), _real_basis(l3)
    W = np.einsum('ai,bj,ck,ijk->abc', U1, U2, U3.conj(), C.astype(np.complex128))
    Wr, Wi = W.real, W.imag
    Wp = Wr if np.linalg.norm(Wr) >= np.linalg.norm(Wi) else Wi
    nrm = np.linalg.norm(Wp)
    if nrm > 0:
        Wp = Wp / nrm
    return Wp.astype(np.float32)


_PATHS = [(l1, l2, l3) for l1 in range(LMAX + 1) for l2 in range(LMAX + 1) for l3 in range(abs(l1 - l2), min(l1 + l2, LMAX) + 1)]
_W3J = [_w3j_real(*p) for p in _PATHS]


def _np_harm(n):
    x, y, z = n[:, 0], n[:, 1], n[:, 2]
    Y0 = np.ones((n.shape[0], 1))
    Y1 = math.sqrt(3.0) * n
    Y2 = np.stack([
        math.sqrt(15.0) * x * y,
        math.sqrt(15.0) * y * z,
        math.sqrt(5.0) / 2.0 * (3.0 * z * z - 1.0),
        math.sqrt(15.0) * x * z,
        math.sqrt(15.0) / 2.0 * (x * x - y * y)], axis=-1)
    return [Y0, Y1, Y2]


def _build_constants():
    rng = np.random.default_rng(12345)
    n = rng.normal(size=(4000, 3))
    n /= np.linalg.norm(n, axis=1, keepdims=True)
    Y = _np_harm(n)
    Z = [np.einsum('ei,ej,ijk->ek', Y[p[0]], Y[p[1]], W.astype(np.float64))
         for p, W in zip(_PATHS, _W3J)]
    cp = {}
    for p in [0, 1, 2, 3, 4, 9, 12, 14]:
        l3 = _PATHS[p][2]
        cp[p] = float((Z[p] * Y[l3]).sum() / (Y[l3] * Y[l3]).sum())
    A6, *_ = np.linalg.lstsq(Y[2], Z[6], rcond=None)
    W7 = _W3J[7].astype(np.float64).reshape(15, 3)
    W8 = _W3J[8].astype(np.float64).reshape(15, 5)
    s8 = 1.0 / math.sqrt(8.0)
    ALPHA = np.zeros((15, 6))
    for p, b in [(0, 0), (4, 0), (12, 0), (1, 1), (3, 1), (2, 3), (9, 3), (14, 3)]:
        ALPHA[p, b] = cp[p] * s8
    ALPHA[7, 2] = s8
    ALPHA[10, 2] = s8
    ALPHA[6, 4] = s8
    ALPHA[8, 5] = s8
    ALPHA[11, 5] = -s8
    WZX = np.zeros((13, 20))
    WZX[0:3, 0:15] = W7.T
    WZX[3:8, 0:15] = W8.T
    WZX[8:13, 15:20] = A6.T
    return ALPHA.astype(np.float32), WZX.astype(np.float32)


_ALPHA, _WZX = _build_constants()
_CENTERS = np.linspace(0.0, MAX_RADIUS, NUM_BASIS + 2)[1:-1].astype(np.float32).reshape(NUM_BASIS, 1)
_RSTEP = float(MAX_RADIUS / (NUM_BASIS + 1))
_EMB_SCALE = float(math.sqrt(NUM_BASIS) / 1.12)


def _silu(x):
    return x * (1.0 / (1.0 + jnp.exp(-x)))


def _prep_body(pos_ref, a_ref, emb_ref, w1_ref, b1_ref, w2_ref, b2_ref, tbl_ref):
    av = a_ref[...]
    io = lax.broadcasted_iota(jnp.int32, (N_PAD, 16), 1)
    oh = jnp.where(io == av, 1.0, 0.0).astype(jnp.float32)
    x = jnp.dot(oh, emb_ref[...], preferred_element_type=jnp.float32)
    h = _silu(jnp.dot(x, w1_ref[...], preferred_element_type=jnp.float32) + b1_ref[...])
    ai = jnp.dot(h, w2_ref[...], preferred_element_type=jnp.float32) + b2_ref[...]
    tbl_ref[...] = jnp.concatenate(
        [pos_ref[...], ai, jnp.zeros((N_PAD, 116), jnp.float32)], axis=1)


def _make_gather(nchunks):
    def _gather_body(tbl_hbm, src_hbm, dst_hbm, osrc_hbm, odst_hbm,
                     idx_s0, idx_d0, row_s0, row_d0, pk_s0, pk_d0,
                     idx_s1, idx_d1, row_s1, row_d1, pk_s1, pk_d1,
                     sem_s0, sem_d0, sem_s1, sem_d1):
        wid = lax.axis_index("s") * 2 + lax.axis_index("c")
        base = wid * (nchunks * CH)
        bufs = ((idx_s0, idx_d0, row_s0, row_d0, pk_s0, pk_d0, sem_s0, sem_d0),
                (idx_s1, idx_d1, row_s1, row_d1, pk_s1, pk_d1, sem_s1, sem_d1))

        def start(i, bs):
            off = base + i * CH
            pltpu.sync_copy(src_hbm.at[pl.ds(off, CH)], bs[0])
            pltpu.sync_copy(dst_hbm.at[pl.ds(off, CH)], bs[1])
            pltpu.async_copy(tbl_hbm.at[bs[0]], bs[2], bs[6])
            pltpu.async_copy(tbl_hbm.at[bs[1]], bs[3], bs[7])

        def finish(i, bs):
            pltpu.make_async_copy(tbl_hbm.at[bs[0]], bs[2], bs[6]).wait()
            pltpu.make_async_copy(tbl_hbm.at[bs[1]], bs[3], bs[7]).wait()
            for e in range(CH):
                j, k = e // 8, e % 8
                bs[4][j, 16 * k:16 * (k + 1)] = bs[2][e, 0:16]
                bs[5][j, 16 * k:16 * (k + 1)] = bs[3][e, 0:16]
            prow = pl.multiple_of((base + i * CH) // 8, 16)
            pltpu.sync_copy(bs[4], osrc_hbm.at[pl.ds(prow, CH // 8)])
            pltpu.sync_copy(bs[5], odst_hbm.at[pl.ds(prow, CH // 8)])

        start(0, bufs[0])

        def body(i2, carry):
            i = i2 * 2

            @pl.when(i + 1 < nchunks)
            def _():
                start(i + 1, bufs[1])

            finish(i, bufs[0])

            @pl.when(i + 2 < nchunks)
            def _():
                start(i + 2, bufs[0])

            @pl.when(i + 1 < nchunks)
            def _():
                finish(i + 1, bufs[1])

            return carry

        lax.fori_loop(0, (nchunks + 1) // 2, body, 0)

    return _gather_body


def _edge_body(s_ref, d_ref, wzx_ref, tmt_ref, w1_ref, b1_ref, w2_ref, b2_ref,
               w3_ref, b3_ref, o_ref):
    B = EB
    St = jnp.concatenate([s_ref[:, 16 * k:16 * k + 12].T for k in range(8)], axis=1)
    Dt = jnp.concatenate([d_ref[:, 16 * k:16 * k + 3].T for k in range(8)], axis=1)
    v3 = Dt[0:3, :] - St[0:3, :]
    ln = jnp.sqrt(v3[0:1, :] * v3[0:1, :] + v3[1:2, :] * v3[1:2, :] + v3[2:3, :] * v3[2:3, :])
    inv = 1.0 / jnp.maximum(ln, 1e-8)
    nv = v3 * inv
    x, y, z = nv[0:1, :], nv[1:2, :], nv[2:3, :]
    s3 = math.sqrt(3.0)
    s15 = math.sqrt(15.0)
    Y1 = s3 * nv
    Y2 = jnp.concatenate([
        s15 * x * y,
        s15 * y * z,
        (math.sqrt(5.0) / 2.0) * (3.0 * z * z - 1.0),
        s15 * x * z,
        (s15 / 2.0) * (x * x - y * y)], axis=0)
    PY = (Y1[:, None, :] * Y2[None, :, :]).reshape(15, B)
    CY = jnp.concatenate([PY, Y2], axis=0)
    ZX = jnp.dot(wzx_ref[...], CY, preferred_element_type=jnp.float32)
    Z7 = ZX[0:3, :]
    Z8 = ZX[3:8, :]
    Z6 = ZX[8:13, :]

    cen = (lax.broadcasted_iota(jnp.int32, (8, 1), 0).astype(jnp.float32) + 1.0) * _RSTEP
    dif = (ln - cen) * (1.0 / _RSTEP)
    emb_t = jnp.exp(-(dif * dif)) * _EMB_SCALE
    emb = emb_t.T
    h1 = _silu(jnp.dot(emb, w1_ref[...], preferred_element_type=jnp.float32) + b1_ref[...])
    h2 = _silu(jnp.dot(h1, w2_ref[...], preferred_element_type=jnp.float32) + b2_ref[...])
    g = jnp.dot(h2, w3_ref[...], preferred_element_type=jnp.float32) + b3_ref[...]
    gt = g.T
    Ait = St[4:12, :]
    X2 = (gt[:, None, :] * Ait[None, :, :]).reshape(120, B)
    m = jnp.dot(tmt_ref[...], X2, preferred_element_type=jnp.float32)

    A1 = (m[8:16, :][:, None, :] * Y1[None, :, :]
          + m[16:24, :][:, None, :] * Z7[None, :, :]).reshape(24, B)
    A2 = (m[24:32, :][:, None, :] * Y2[None, :, :]
          + m[32:40, :][:, None, :] * Z6[None, :, :]
          + m[40:48, :][:, None, :] * Z8[None, :, :]).reshape(40, B)
    EF = jnp.concatenate([m[0:8, :], A1, A2], axis=0)
    o_ref[:, 0:72] = EF.T


def _make_scatter(nchunks):
    def _scatter_body(dst_hbm, ef_hbm, zer_hbm, out_hbm,
                      idx_v0, row_v0, idx_v1, row_v1, acc, sem0, sem1):
        cid = lax.axis_index("c")
        sid = lax.axis_index("s")
        wid = sid * 2 + cid
        base = wid * (nchunks * CH)
        bufs = ((idx_v0, row_v0, sem0), (idx_v1, row_v1, sem1))
        pltpu.sync_copy(zer_hbm.at[pl.ds(sid * ROWS_W, ROWS_W)],
                        acc.at[pl.ds(sid * ROWS_W, ROWS_W)])
        plsc.subcore_barrier()

        def start(i, bs):
            off = base + i * CH
            pltpu.sync_copy(dst_hbm.at[pl.ds(off, CH)], bs[0])
            pltpu.async_copy(ef_hbm.at[pl.ds(off, CH)], bs[1], bs[2])

        def finish(i, bs):
            off = base + i * CH
            pltpu.make_async_copy(ef_hbm.at[pl.ds(off, CH)], bs[1], bs[2]).wait()
            pltpu.sync_copy(bs[1], acc.at[bs[0]], add=True)

        start(0, bufs[0])

        def body(i2, carry):
            i = i2 * 2

            @pl.when(i + 1 < nchunks)
            def _():
                start(i + 1, bufs[1])

            finish(i, bufs[0])

            @pl.when(i + 2 < nchunks)
            def _():
                start(i + 2, bufs[0])

            @pl.when(i + 1 < nchunks)
            def _():
                finish(i + 1, bufs[1])

            return carry

        lax.fori_loop(0, (nchunks + 1) // 2, body, 0)
        plsc.subcore_barrier()
        pltpu.sync_copy(acc.at[pl.ds(sid * ROWS_W, ROWS_W)],
                        out_hbm.at[cid].at[pl.ds(sid * ROWS_W, ROWS_W)])

    return _scatter_body


def _combine_body(*refs):
    parts, o_ref = refs[:-1], refs[-1]
    s = parts[0][0, :, 0:72] + parts[0][1, :, 0:72]
    for p_ref in parts[1:]:
        s = s + p_ref[0, :, 0:72] + p_ref[1, :, 0:72]
    o_ref[...] = s * (1.0 / AVG_NEIGH)


def kernel(pos, A, batch, edge_src, edge_dst, edge_shifts, cell, emb_table,
           w1, b1, w2, b2, fc_w1, fc_b1, fc_w2, fc_b2, fc_w3, fc_b3, tp_weights):
    f32 = jnp.float32
    pos_p = jnp.concatenate([pos.astype(f32), jnp.zeros((N_PAD - N_NODES, 3), f32)], axis=0)
    pos_p4 = jnp.concatenate([pos_p, jnp.zeros((N_PAD, 1), f32)], axis=1)
    a_p = jnp.concatenate([A.astype(jnp.int32), jnp.zeros((N_PAD - N_NODES,), jnp.int32)]).reshape(N_PAD, 1)
    emb_p = jnp.concatenate([emb_table.astype(f32),
                             jnp.zeros((16 - emb_table.shape[0], 16), f32)], axis=0)
    src_p = jnp.concatenate([edge_src.astype(jnp.int32),
                             jnp.zeros((E_PAD - N_EDGES,), jnp.int32)])
    dst_p = jnp.concatenate([edge_dst.astype(jnp.int32),
                             jnp.full((E_PAD - N_EDGES,), N_PAD - 1, jnp.int32)])
    dst_perm = dst_p.reshape(E_PAD // EB, EB // CH, 16, 8).transpose(0, 3, 1, 2).reshape(-1)
    tmt = jnp.einsum('pb,puv->bvpu', jnp.asarray(_ALPHA), tp_weights.astype(f32)).reshape(48, 120)
    wzx = jnp.asarray(_WZX)
    zer = jnp.zeros((N_PAD, 128), f32)

    tbl = pl.pallas_call(
        _prep_body,
        out_shape=jax.ShapeDtypeStruct((N_PAD, 128), f32),
    )(pos_p4, a_p, emb_p, w1.astype(f32), b1.reshape(1, 64).astype(f32),
      w2.astype(f32), b2.reshape(1, 8).astype(f32))

    mesh = plsc.VectorSubcoreMesh(core_axis_name="c", subcore_axis_name="s")
    partials = []
    b0 = 0
    for nb in PART_BLOCKS:
        ne = nb * EB
        e0 = b0 * EB
        gath = functools.partial(
            pl.kernel, mesh=mesh,
            out_type=[jax.ShapeDtypeStruct((ne // 8, 128), f32),
                      jax.ShapeDtypeStruct((ne // 8, 128), f32)],
            scratch_types=[pltpu.VMEM((CH,), jnp.int32), pltpu.VMEM((CH,), jnp.int32),
                           pltpu.VMEM((CH, 128), f32), pltpu.VMEM((CH, 128), f32),
                           pltpu.VMEM((CH // 8, 128), f32), pltpu.VMEM((CH // 8, 128), f32),
                           pltpu.VMEM((CH,), jnp.int32), pltpu.VMEM((CH,), jnp.int32),
                           pltpu.VMEM((CH, 128), f32), pltpu.VMEM((CH, 128), f32),
                           pltpu.VMEM((CH // 8, 128), f32), pltpu.VMEM((CH // 8, 128), f32),
                           pltpu.SemaphoreType.DMA, pltpu.SemaphoreType.DMA,
                           pltpu.SemaphoreType.DMA, pltpu.SemaphoreType.DMA],
        )(_make_gather(nb))
        srows, drows = gath(tbl, lax.dynamic_slice(src_p, (e0,), (ne,)),
                            lax.dynamic_slice(dst_p, (e0,), (ne,)))

        ef = pl.pallas_call(
            _edge_body,
            grid=(nb,),
            in_specs=[
                pl.BlockSpec((EB // 8, 128), lambda i: (i, 0)),
                pl.BlockSpec((EB // 8, 128), lambda i: (i, 0)),
                pl.BlockSpec((13, 20), lambda i: (0, 0)),
                pl.BlockSpec((48, 120), lambda i: (0, 0)),
                pl.BlockSpec((8, 64), lambda i: (0, 0)),
                pl.BlockSpec((1, 64), lambda i: (0, 0)),
                pl.BlockSpec((64, 64), lambda i: (0, 0)),
                pl.BlockSpec((1, 64), lambda i: (0, 0)),
                pl.BlockSpec((64, 15), lambda i: (0, 0)),
                pl.BlockSpec((1, 15), lambda i: (0, 0)),
            ],
            out_specs=pl.BlockSpec((EB, 128), lambda i: (i, 0)),
            out_shape=jax.ShapeDtypeStruct((ne, 128), f32),
        )(srows, drows, wzx, tmt,
          fc_w1.astype(f32), fc_b1.reshape(1, 64).astype(f32),
          fc_w2.astype(f32), fc_b2.reshape(1, 64).astype(f32),
          fc_w3.astype(f32), fc_b3.reshape(1, 15).astype(f32))

        scat = functools.partial(
            pl.kernel, mesh=mesh,
            out_type=jax.ShapeDtypeStruct((2, N_PAD, 128), f32),
            scratch_types=[pltpu.VMEM((CH,), jnp.int32), pltpu.VMEM((CH, 128), f32),
                           pltpu.VMEM((CH,), jnp.int32), pltpu.VMEM((CH, 128), f32),
                           pltpu.VMEM_SHARED((N_PAD, 128), f32),
                           pltpu.SemaphoreType.DMA, pltpu.SemaphoreType.DMA],
        )(_make_scatter(nb))
        partials.append(scat(lax.dynamic_slice(dst_perm, (e0,), (ne,)), ef, zer))
        b0 += nb

    out = pl.pallas_call(
        _combine_body,
        grid=(5,),
        in_specs=[pl.BlockSpec((2, N_NODES // 5, 128), lambda i: (0, i, 0))
                  for _ in partials],
        out_specs=pl.BlockSpec((N_NODES // 5, 72), lambda i: (i, 0)),
        out_shape=jax.ShapeDtypeStruct((N_NODES, 72), f32),
    )(*partials)
    return out
